# baseline (speedup 1.0000x reference)
import functools

import jax
import jax.numpy as jnp
from jax import lax
from jax.experimental import pallas as pl
from jax.experimental.pallas import tpu as pltpu
from jax.experimental.pallas import tpu_sc as plsc

_TOPK = 10
_NC = 16
_IMAX = jnp.iinfo(jnp.int32).max
_LANES = 16
_CHUNKC = 2048
_NQ = 4
_RT = 8


def _merge16(tv, ti, v, vi):
    sa, ia = plsc.sort_key_val(tv, ti, descending=True)
    sb, ib = plsc.sort_key_val(v, vi, descending=True)
    rb = lax.rev(sb, (0,))
    rib = lax.rev(ib, (0,))
    take = sa >= rb
    return jnp.where(take, sa, rb), jnp.where(take, ia, rib)


def _scan_row_block(buf, r8, tv, ti, thrv, colbase, nvreg, lane):
    subs = []
    for s in range(0, nvreg, 16):
        cnt = min(16, nvreg - s)
        acc = None
        for u in range(s, s + cnt):
            v = buf[r8, pl.ds(u * _LANES, _LANES)]
            acc = v if acc is None else jnp.maximum(acc, v)
        subs.append((s, cnt, acc))
    g = subs[0][2]
    for _, _, a in subs[1:]:
        g = jnp.maximum(g, a)
    pred = plsc.all_reduce_population_count(g > thrv)[0] > 0

    def insert(args):
        tv, ti, thr = args
        for s, cnt, a in subs:
            sgo = plsc.all_reduce_population_count(a > thr)[0] > 0

            def ins_sub(args2, s=s, cnt=cnt):
                tv, ti, thr = args2
                macc = m2acc = None
                iacc = None
                for u in range(s, s + cnt):
                    v = buf[r8, pl.ds(u * _LANES, _LANES)]
                    vi = lane + (colbase + u * _LANES)
                    if macc is None:
                        macc, iacc = v, vi
                        m2acc = jnp.full((_LANES,), -jnp.inf, jnp.float32)
                    else:
                        big = v > macc
                        m2acc = jnp.maximum(m2acc, jnp.where(big, macc, v))
                        macc = jnp.where(big, v, macc)
                        iacc = jnp.where(big, vi, iacc)
                tv, ti = _merge16(tv, ti, macc, iacc)
                thr = -jnp.max(-tv)

                def ins_rest(args3, s=s, cnt=cnt, m2acc=m2acc, iacc=iacc):
                    tv, ti, thr = args3
                    for u in range(s, s + cnt):
                        v = buf[r8, pl.ds(u * _LANES, _LANES)]
                        vi = lane + (colbase + u * _LANES)
                        vm = jnp.where(vi != iacc, v, -jnp.inf)
                        vgo = plsc.all_reduce_population_count(
                            vm > thr)[0] > 0

                        def ins_vreg(args4, vm=vm, vi=vi):
                            tv, ti, thr = args4
                            tv, ti = _merge16(tv, ti, vm, vi)
                            return tv, ti, thr

                        tv, ti, thr = lax.cond(vgo, ins_vreg,
                                               lambda x: x, (tv, ti, thr))
                    tv2, ti2 = tv, ti
                    return tv2, ti2, -jnp.max(-tv2)

                pred2 = plsc.all_reduce_population_count(
                    m2acc > thr)[0] > 0
                return lax.cond(pred2, ins_rest, lambda x: x, (tv, ti, thr))

            tv, ti, thr = lax.cond(sgo, ins_sub, lambda x: x,
                                   (tv, ti, thr))
        return tv, ti, thr

    return lax.cond(pred, insert, lambda x: x, (tv, ti, thrv[0]))


def _sc_body(vocab, logits_hbm, outv_hbm, outi_hbm, buf0, buf1, tailbuf,
             tvs, tis, thrs, sem0, sem1):
    wid = lax.axis_index("s") * 2 + lax.axis_index("c")
    rt = wid // _NQ
    q = wid % _NQ
    row0 = pl.multiple_of(rt * _RT, 8)
    lane = lax.broadcasted_iota(jnp.int32, (_LANES,), 0)

    nmain = vocab // (_NQ * _CHUNKC)
    tail0 = nmain * _NQ * _CHUNKC
    ntail = vocab - tail0

    for r8 in range(_RT):
        tvs[r8] = jnp.full((_LANES,), -jnp.inf, jnp.float32)
        tis[r8] = jnp.zeros((_LANES,), jnp.int32)
        thrs[r8] = jnp.full((_LANES,), -jnp.inf, jnp.float32)

    def src(c):
        colbase = (q * nmain + c) * _CHUNKC
        return logits_hbm.at[pl.ds(row0, _RT),
                             pl.ds(pl.multiple_of(colbase, 128), _CHUNKC)]

    def rows(c, buf):
        colbase = (q * nmain + c) * _CHUNKC

        def row_body(r8, __):
            tv, ti, thr = _scan_row_block(buf, r8, tvs[r8], tis[r8],
                                          thrs[r8], colbase,
                                          _CHUNKC // _LANES, lane)
            tvs[r8] = tv
            tis[r8] = ti
            thrs[r8] = jnp.broadcast_to(thr, (_LANES,))
            return 0

        lax.fori_loop(0, _RT, row_body, 0)

    pltpu.async_copy(src(0), buf0, sem0)

    def pair_body(c2, _):
        c0 = 2 * c2
        pltpu.make_async_copy(src(0), buf0, sem0).wait()
        pltpu.async_copy(src(c0 + 1), buf1, sem1)
        rows(c0, buf0)
        pltpu.make_async_copy(src(0), buf1, sem1).wait()
        pltpu.async_copy(src(jnp.minimum(c0 + 2, nmain - 1)), buf0, sem0)
        rows(c0 + 1, buf1)
        return 0

    lax.fori_loop(0, nmain // 2, pair_body, 0)
    pltpu.make_async_copy(src(0), buf0, sem0).wait()

    if ntail:
        pltpu.sync_copy(
            logits_hbm.at[pl.ds(row0, _RT), pl.ds(tail0, ntail)], tailbuf)

        def tail_row(r8, __):
            tv, ti, thr = _scan_row_block(tailbuf, r8, tvs[r8], tis[r8],
                                          thrs[r8], tail0,
                                          ntail // _LANES, lane)
            tvs[r8] = tv
            tis[r8] = ti
            thrs[r8] = jnp.broadcast_to(thr, (_LANES,))
            return 0

        lax.fori_loop(0, _RT, tail_row, 0)

    out_off = pl.multiple_of(q * 64 + row0, 8)
    pltpu.sync_copy(tvs, outv_hbm.at[pl.ds(out_off, _RT)])
    pltpu.sync_copy(tis, outi_hbm.at[pl.ds(out_off, _RT)])


def _sample_body(v_ref, i_ref, g_ref, o_ref):
    rows = g_ref.shape[0]
    width = _NQ * _NC
    lane = lax.broadcasted_iota(jnp.int32, (rows, width), 1)
    cv = lax.concatenate(
        [v_ref[q * rows:(q + 1) * rows, :] for q in range(_NQ)], 1)
    ci = lax.concatenate(
        [i_ref[q * rows:(q + 1) * rows, :] for q in range(_NQ)], 1)
    nvv = jnp.full((rows, width), -jnp.inf, jnp.float32)
    nii = jnp.zeros((rows, width), jnp.int32)
    for t in range(_TOPK):
        m = jnp.max(cv, axis=1, keepdims=True)
        j = jnp.min(jnp.where(cv == m, ci, _IMAX), axis=1, keepdims=True)
        sel = lane == t
        nvv = jnp.where(sel, m, nvv)
        nii = jnp.where(sel, j, nii)
        cv = jnp.where(ci == j, -jnp.inf, cv)
    m = jnp.max(nvv, axis=1, keepdims=True)
    u = jnp.exp(nvv - m)
    p = u / jnp.sum(u, axis=1, keepdims=True)
    t_ = jnp.log(p + 1e-20) + g_ref[...]
    tm = jnp.max(t_, axis=1, keepdims=True)
    spos = jnp.min(jnp.where(t_ == tm, lane, _IMAX), axis=1, keepdims=True)
    o_ref[...] = jnp.min(jnp.where(lane == spos, nii, _IMAX), axis=1,
                         keepdims=True)


def kernel(logits):
    rows, vocab = logits.shape

    mesh = plsc.VectorSubcoreMesh(core_axis_name="c", subcore_axis_name="s")
    sc_topk = pl.kernel(
        functools.partial(_sc_body, vocab),
        out_type=[jax.ShapeDtypeStruct((_NQ * rows, _NC), jnp.float32),
                  jax.ShapeDtypeStruct((_NQ * rows, _NC), jnp.int32)],
        mesh=mesh,
        scratch_types=[pltpu.VMEM((_RT, _CHUNKC), jnp.float32),
                       pltpu.VMEM((_RT, _CHUNKC), jnp.float32),
                       pltpu.VMEM((_RT, 576), jnp.float32),
                       pltpu.VMEM((_RT, _NC), jnp.float32),
                       pltpu.VMEM((_RT, _NC), jnp.int32),
                       pltpu.VMEM((_RT, _LANES), jnp.float32),
                       pltpu.SemaphoreType.DMA,
                       pltpu.SemaphoreType.DMA],
        compiler_params=pltpu.CompilerParams(needs_layout_passes=False),
    )
    cand_v, cand_i = sc_topk(logits)

    g = jax.random.gumbel(jax.random.key(42), (rows, _TOPK), jnp.float32)
    gpad = jnp.full((rows, _NQ * _NC), -jnp.inf, jnp.float32)
    gpad = gpad.at[:, :_TOPK].set(g)

    return pl.pallas_call(
        _sample_body,
        in_specs=[pl.BlockSpec((_NQ * rows, _NC), lambda: (0, 0)),
                  pl.BlockSpec((_NQ * rows, _NC), lambda: (0, 0)),
                  pl.BlockSpec((rows, _NQ * _NC), lambda: (0, 0))],
        out_specs=pl.BlockSpec((rows, 1), lambda: (0, 0)),
        out_shape=jax.ShapeDtypeStruct((rows, 1), jnp.int32),
    )(cand_v, cand_i, gpad)


# speedup vs baseline: 2.9382x; 2.9382x over previous
import functools

import jax
import jax.numpy as jnp
from jax import lax
from jax.experimental import pallas as pl
from jax.experimental.pallas import tpu as pltpu
from jax.experimental.pallas import tpu_sc as plsc

_TOPK = 10
_NC = 16
_IMAX = jnp.iinfo(jnp.int32).max
_LANES = 16
_CHUNKC = 512
_NQ = 4
_RT = 8


def _merge16(tv, ti, v, vi):
    sa, ia = plsc.sort_key_val(tv, ti, descending=True)
    sb, ib = plsc.sort_key_val(v, vi, descending=True)
    rb = lax.rev(sb, (0,))
    rib = lax.rev(ib, (0,))
    take = sa >= rb
    return jnp.where(take, sa, rb), jnp.where(take, ia, rib)


def _scan_row_block(buf, r8, tv, ti, thrv, colbase, nvreg, lane):
    subs = []
    for s in range(0, nvreg, 16):
        cnt = min(16, nvreg - s)
        acc = None
        for u in range(s, s + cnt):
            v = buf[r8, pl.ds(u * _LANES, _LANES)]
            acc = v if acc is None else jnp.maximum(acc, v)
        subs.append((s, cnt, acc))
    g = subs[0][2]
    for _, _, a in subs[1:]:
        g = jnp.maximum(g, a)
    pred = plsc.all_reduce_population_count(g > thrv)[0] > 0

    def insert(args):
        tv, ti, thr = args
        for s, cnt, a in subs:
            sgo = plsc.all_reduce_population_count(a > thr)[0] > 0

            def ins_sub(args2, s=s, cnt=cnt):
                tv, ti, thr = args2
                macc = m2acc = None
                iacc = None
                for u in range(s, s + cnt):
                    v = buf[r8, pl.ds(u * _LANES, _LANES)]
                    vi = lane + (colbase + u * _LANES)
                    if macc is None:
                        macc, iacc = v, vi
                        m2acc = jnp.full((_LANES,), -jnp.inf, jnp.float32)
                    else:
                        big = v > macc
                        m2acc = jnp.maximum(m2acc, jnp.where(big, macc, v))
                        macc = jnp.where(big, v, macc)
                        iacc = jnp.where(big, vi, iacc)
                tv, ti = _merge16(tv, ti, macc, iacc)
                thr = -jnp.max(-tv)

                def ins_rest(args3, s=s, cnt=cnt, m2acc=m2acc, iacc=iacc):
                    tv, ti, thr = args3
                    for u in range(s, s + cnt):
                        v = buf[r8, pl.ds(u * _LANES, _LANES)]
                        vi = lane + (colbase + u * _LANES)
                        vm = jnp.where(vi != iacc, v, -jnp.inf)
                        vgo = plsc.all_reduce_population_count(
                            vm > thr)[0] > 0

                        def ins_vreg(args4, vm=vm, vi=vi):
                            tv, ti, thr = args4
                            tv, ti = _merge16(tv, ti, vm, vi)
                            return tv, ti, thr

                        tv, ti, thr = lax.cond(vgo, ins_vreg,
                                               lambda x: x, (tv, ti, thr))
                    tv2, ti2 = tv, ti
                    return tv2, ti2, -jnp.max(-tv2)

                pred2 = plsc.all_reduce_population_count(
                    m2acc > thr)[0] > 0
                return lax.cond(pred2, ins_rest, lambda x: x, (tv, ti, thr))

            tv, ti, thr = lax.cond(sgo, ins_sub, lambda x: x,
                                   (tv, ti, thr))
        return tv, ti, thr

    return lax.cond(pred, insert, lambda x: x, (tv, ti, thrv[0]))


def _sc_body(vocab, logits_hbm, outv_hbm, outi_hbm, buf0, buf1, tailbuf,
             tvs, tis, thrs, sem0, sem1):
    wid = lax.axis_index("s") * 2 + lax.axis_index("c")
    rt = wid // _NQ
    q = wid % _NQ
    row0 = pl.multiple_of(rt * _RT, 8)
    lane = lax.broadcasted_iota(jnp.int32, (_LANES,), 0)

    nmain = vocab // (_NQ * _CHUNKC)
    tail0 = nmain * _NQ * _CHUNKC
    ntail = vocab - tail0

    for r8 in range(_RT):
        tvs[r8] = jnp.full((_LANES,), -jnp.inf, jnp.float32)
        tis[r8] = jnp.zeros((_LANES,), jnp.int32)
        thrs[r8] = jnp.full((_LANES,), -jnp.inf, jnp.float32)

    def src(c):
        colbase = (q * nmain + c) * _CHUNKC
        return logits_hbm.at[pl.ds(row0, _RT),
                             pl.ds(pl.multiple_of(colbase, 128), _CHUNKC)]

    def rows(c, buf):
        colbase = (q * nmain + c) * _CHUNKC

        def row_body(r8, __):
            tv, ti, thr = _scan_row_block(buf, r8, tvs[r8], tis[r8],
                                          thrs[r8], colbase,
                                          _CHUNKC // _LANES, lane)
            tvs[r8] = tv
            tis[r8] = ti
            thrs[r8] = jnp.broadcast_to(thr, (_LANES,))
            return 0

        lax.fori_loop(0, _RT, row_body, 0)

    pltpu.async_copy(src(0), buf0, sem0)

    def pair_body(c2, _):
        c0 = 2 * c2
        pltpu.make_async_copy(src(0), buf0, sem0).wait()
        pltpu.async_copy(src(c0 + 1), buf1, sem1)
        rows(c0, buf0)
        pltpu.make_async_copy(src(0), buf1, sem1).wait()
        pltpu.async_copy(src(jnp.minimum(c0 + 2, nmain - 1)), buf0, sem0)
        rows(c0 + 1, buf1)
        return 0

    lax.fori_loop(0, nmain // 2, pair_body, 0)
    pltpu.make_async_copy(src(0), buf0, sem0).wait()

    if ntail:
        pltpu.sync_copy(
            logits_hbm.at[pl.ds(row0, _RT), pl.ds(tail0, ntail)], tailbuf)

        def tail_row(r8, __):
            tv, ti, thr = _scan_row_block(tailbuf, r8, tvs[r8], tis[r8],
                                          thrs[r8], tail0,
                                          ntail // _LANES, lane)
            tvs[r8] = tv
            tis[r8] = ti
            thrs[r8] = jnp.broadcast_to(thr, (_LANES,))
            return 0

        lax.fori_loop(0, _RT, tail_row, 0)

    out_off = pl.multiple_of(q * 64 + row0, 8)
    pltpu.sync_copy(tvs, outv_hbm.at[pl.ds(out_off, _RT)])
    pltpu.sync_copy(tis, outi_hbm.at[pl.ds(out_off, _RT)])


def _sample_body(v_ref, i_ref, g_ref, o_ref):
    rows = g_ref.shape[0]
    width = _NQ * _NC
    lane = lax.broadcasted_iota(jnp.int32, (rows, width), 1)
    cv = lax.concatenate(
        [v_ref[q * rows:(q + 1) * rows, :] for q in range(_NQ)], 1)
    ci = lax.concatenate(
        [i_ref[q * rows:(q + 1) * rows, :] for q in range(_NQ)], 1)
    nvv = jnp.full((rows, width), -jnp.inf, jnp.float32)
    nii = jnp.zeros((rows, width), jnp.int32)
    for t in range(_TOPK):
        m = jnp.max(cv, axis=1, keepdims=True)
        j = jnp.min(jnp.where(cv == m, ci, _IMAX), axis=1, keepdims=True)
        sel = lane == t
        nvv = jnp.where(sel, m, nvv)
        nii = jnp.where(sel, j, nii)
        cv = jnp.where(ci == j, -jnp.inf, cv)
    m = jnp.max(nvv, axis=1, keepdims=True)
    u = jnp.exp(nvv - m)
    p = u / jnp.sum(u, axis=1, keepdims=True)
    t_ = jnp.log(p + 1e-20) + g_ref[...]
    tm = jnp.max(t_, axis=1, keepdims=True)
    spos = jnp.min(jnp.where(t_ == tm, lane, _IMAX), axis=1, keepdims=True)
    o_ref[...] = jnp.min(jnp.where(lane == spos, nii, _IMAX), axis=1,
                         keepdims=True)


def kernel(logits):
    rows, vocab = logits.shape

    mesh = plsc.VectorSubcoreMesh(core_axis_name="c", subcore_axis_name="s")
    sc_topk = pl.kernel(
        functools.partial(_sc_body, vocab),
        out_type=[jax.ShapeDtypeStruct((_NQ * rows, _NC), jnp.float32),
                  jax.ShapeDtypeStruct((_NQ * rows, _NC), jnp.int32)],
        mesh=mesh,
        scratch_types=[pltpu.VMEM((_RT, _CHUNKC), jnp.float32),
                       pltpu.VMEM((_RT, _CHUNKC), jnp.float32),
                       pltpu.VMEM((_RT, 576), jnp.float32),
                       pltpu.VMEM((_RT, _NC), jnp.float32),
                       pltpu.VMEM((_RT, _NC), jnp.int32),
                       pltpu.VMEM((_RT, _LANES), jnp.float32),
                       pltpu.SemaphoreType.DMA,
                       pltpu.SemaphoreType.DMA],
        compiler_params=pltpu.CompilerParams(needs_layout_passes=False),
    )
    cand_v, cand_i = sc_topk(logits)

    g = jax.random.gumbel(jax.random.key(42), (rows, _TOPK), jnp.float32)
    gpad = jnp.full((rows, _NQ * _NC), -jnp.inf, jnp.float32)
    gpad = gpad.at[:, :_TOPK].set(g)

    return pl.pallas_call(
        _sample_body,
        in_specs=[pl.BlockSpec((_NQ * rows, _NC), lambda: (0, 0)),
                  pl.BlockSpec((_NQ * rows, _NC), lambda: (0, 0)),
                  pl.BlockSpec((rows, _NQ * _NC), lambda: (0, 0))],
        out_specs=pl.BlockSpec((rows, 1), lambda: (0, 0)),
        out_shape=jax.ShapeDtypeStruct((rows, 1), jnp.int32),
    )(cand_v, cand_i, gpad)
